# baseline (speedup 1.0000x reference)
import functools

import jax
import jax.numpy as jnp
from jax import lax
from jax.experimental import pallas as pl
from jax.experimental.pallas import tpu as pltpu
from jax.experimental.pallas import tpu_sc as plsc

_N = 16384
_LANES = 16

_info = plsc.get_sparse_core_info()
_NC = _info.num_cores
_NS = _info.num_subcores
_NW = _NC * _NS
_CHUNK = _N // _NW


def _lookup_body(in_hbm, out_hbm, buf):
    wid = lax.axis_index("s") * _NC + lax.axis_index("c")
    base = wid * _CHUNK
    pltpu.sync_copy(in_hbm.at[pl.ds(base, _CHUNK)], buf)
    for i in range(_CHUNK // _LANES):
        x = buf[pl.ds(i * _LANES, _LANES)]
        hit = (x >= 1) & (x <= 4)
        buf[pl.ds(i * _LANES, _LANES)] = jnp.where(hit, 5 - x, -1)
    pltpu.sync_copy(buf, out_hbm.at[pl.ds(base, _CHUNK)])


@jax.jit
def kernel(input):
    x = input.astype(jnp.int32)
    sc_call = pl.kernel(
        _lookup_body,
        out_type=jax.ShapeDtypeStruct((_N,), jnp.int32),
        mesh=plsc.VectorSubcoreMesh(core_axis_name="c", subcore_axis_name="s"),
        scratch_types=[pltpu.VMEM((_CHUNK,), jnp.int32)],
    )
    out = sc_call(x)
    return out.astype(input.dtype)


# speedup vs baseline: 1.0138x; 1.0138x over previous
import functools

import jax
import jax.numpy as jnp
from jax import lax
from jax.experimental import pallas as pl
from jax.experimental.pallas import tpu as pltpu
from jax.experimental.pallas import tpu_sc as plsc

_N = 16384
_LANES = 16

_info = plsc.get_sparse_core_info()
_NC = _info.num_cores
_NS = _info.num_subcores
_NW = _NC * _NS
_CHUNK = _N // _NW


def _lookup_body(in_hbm, out_hbm, buf):
    wid = lax.axis_index("s") * _NC + lax.axis_index("c")
    base = wid * _CHUNK
    pltpu.sync_copy(in_hbm.at[pl.ds(base, _CHUNK)], buf)

    def step(i, carry):
        x = buf[pl.ds(i * _LANES, _LANES)]
        hit = (x >= 1) & (x <= 4)
        buf[pl.ds(i * _LANES, _LANES)] = jnp.where(hit, 5 - x, -1)
        return carry

    lax.fori_loop(0, _CHUNK // _LANES, step, 0, unroll=4)
    pltpu.sync_copy(buf, out_hbm.at[pl.ds(base, _CHUNK)])


@jax.jit
def kernel(input):
    x = input.astype(jnp.int32)
    sc_call = pl.kernel(
        _lookup_body,
        out_type=jax.ShapeDtypeStruct((_N,), jnp.int32),
        mesh=plsc.VectorSubcoreMesh(core_axis_name="c", subcore_axis_name="s"),
        scratch_types=[pltpu.VMEM((_CHUNK,), jnp.int32)],
    )
    out = sc_call(x)
    return out.astype(input.dtype)


# speedup vs baseline: 1.1090x; 1.0939x over previous
import functools

import jax
import jax.numpy as jnp
from jax import lax
from jax.experimental import pallas as pl
from jax.experimental.pallas import tpu as pltpu
from jax.experimental.pallas import tpu_sc as plsc

_N = 16384
_LANES = 16

_info = plsc.get_sparse_core_info()
_NC = 1
_NS = _info.num_subcores
_NW = _NC * _NS
_CHUNK = _N // _NW


def _lookup_body(in_hbm, out_hbm, buf):
    wid = lax.axis_index("s") * _NC + lax.axis_index("c")
    base = wid * _CHUNK
    pltpu.sync_copy(in_hbm.at[pl.ds(base, _CHUNK)], buf)

    def step(i, carry):
        x = buf[pl.ds(i * _LANES, _LANES)]
        hit = (x >= 1) & (x <= 4)
        buf[pl.ds(i * _LANES, _LANES)] = jnp.where(hit, 5 - x, -1)
        return carry

    lax.fori_loop(0, _CHUNK // _LANES, step, 0, unroll=4)
    pltpu.sync_copy(buf, out_hbm.at[pl.ds(base, _CHUNK)])


@jax.jit
def kernel(input):
    x = input.astype(jnp.int32)
    sc_call = pl.kernel(
        _lookup_body,
        out_type=jax.ShapeDtypeStruct((_N,), jnp.int32),
        mesh=plsc.VectorSubcoreMesh(
            core_axis_name="c", subcore_axis_name="s", num_cores=_NC
        ),
        scratch_types=[pltpu.VMEM((_CHUNK,), jnp.int32)],
    )
    out = sc_call(x)
    return out.astype(input.dtype)
